# Initial kernel scaffold; baseline (speedup 1.0000x reference)
import jax
import jax.numpy as jnp
from jax.experimental import pallas as pl


def kernel(input, W):
    raise NotImplementedError("write your pallas kernel here")


# speedup vs baseline: 1.0327x; 1.0327x over previous
import functools

import jax
import jax.numpy as jnp
from jax.experimental import pallas as pl

_NUM_EXPERTS = 64
_TOP_K = 8
_TILE = 512


def _router_body(x_ref, w_ref, probs_ref, idx_ref):
    x = x_ref[...]
    w = w_ref[...]
    logits = jax.lax.dot_general(
        x, w, (((1,), (1,)), ((), ())), preferred_element_type=jnp.float32
    )

    cols = jax.lax.broadcasted_iota(jnp.int32, logits.shape, 1)
    work = logits
    vals = []
    idxs = []
    for _ in range(_TOP_K):
        m = jnp.max(work, axis=1, keepdims=True)
        sel = jnp.min(
            jnp.where(work == m, cols, _NUM_EXPERTS), axis=1, keepdims=True
        )
        vals.append(m)
        idxs.append(sel)
        work = jnp.where(cols == sel, jnp.float32(-jnp.inf), work)

    top = jnp.concatenate(vals, axis=1)
    e = jnp.exp(top - top[:, 0:1])
    probs_ref[...] = e / jnp.sum(e, axis=1, keepdims=True)
    idx_ref[...] = jnp.concatenate(idxs, axis=1)


@jax.jit
def kernel(input, W):
    n_tokens, d = input.shape
    n_exp = W.shape[0]
    grid = n_tokens // _TILE
    probs, indices = pl.pallas_call(
        _router_body,
        grid=(grid,),
        in_specs=[
            pl.BlockSpec((_TILE, d), lambda i: (i, 0)),
            pl.BlockSpec((n_exp, d), lambda i: (0, 0)),
        ],
        out_specs=[
            pl.BlockSpec((_TILE, _TOP_K), lambda i: (i, 0)),
            pl.BlockSpec((_TILE, _TOP_K), lambda i: (i, 0)),
        ],
        out_shape=[
            jax.ShapeDtypeStruct((n_tokens, _TOP_K), jnp.float32),
            jax.ShapeDtypeStruct((n_tokens, _TOP_K), jnp.int32),
        ],
    )(input, W)
    return probs, indices


# speedup vs baseline: 1.4591x; 1.4129x over previous
import functools

import jax
import jax.numpy as jnp
from jax.experimental import pallas as pl

_NUM_EXPERTS = 64
_TOP_K = 8
_TILE = 512


def _router_body(x_ref, w_ref, probs_ref, idx_ref):
    x = x_ref[...]
    w = w_ref[...]
    logits_t = jax.lax.dot_general(
        w, x, (((1,), (1,)), ((), ())), preferred_element_type=jnp.float32
    )

    rows = jax.lax.broadcasted_iota(jnp.int32, logits_t.shape, 0)
    work = logits_t
    vals = []
    idxs = []
    for _ in range(_TOP_K):
        m = jnp.max(work, axis=0, keepdims=True)
        sel = jnp.min(
            jnp.where(work == m, rows, _NUM_EXPERTS), axis=0, keepdims=True
        )
        vals.append(m)
        idxs.append(sel)
        work = jnp.where(rows == sel, jnp.float32(-jnp.inf), work)

    top = jnp.concatenate(vals, axis=0)
    e = jnp.exp(top - top[0:1, :])
    probs_ref[...] = jnp.transpose(e / jnp.sum(e, axis=0, keepdims=True))
    idx_ref[...] = jnp.transpose(jnp.concatenate(idxs, axis=0))


@jax.jit
def kernel(input, W):
    n_tokens, d = input.shape
    n_exp = W.shape[0]
    grid = n_tokens // _TILE
    probs, indices = pl.pallas_call(
        _router_body,
        grid=(grid,),
        in_specs=[
            pl.BlockSpec((_TILE, d), lambda i: (i, 0)),
            pl.BlockSpec((n_exp, d), lambda i: (0, 0)),
        ],
        out_specs=[
            pl.BlockSpec((_TILE, _TOP_K), lambda i: (i, 0)),
            pl.BlockSpec((_TILE, _TOP_K), lambda i: (i, 0)),
        ],
        out_shape=[
            jax.ShapeDtypeStruct((n_tokens, _TOP_K), jnp.float32),
            jax.ShapeDtypeStruct((n_tokens, _TOP_K), jnp.int32),
        ],
    )(input, W)
    return probs, indices


# speedup vs baseline: 1.5727x; 1.0778x over previous
import functools

import jax
import jax.numpy as jnp
from jax.experimental import pallas as pl

_NUM_EXPERTS = 64
_TOP_K = 8
_TILE = 1024


def _router_body(x_ref, w_ref, probs_ref, idx_ref):
    x = x_ref[...]
    w = w_ref[...]
    logits_t = jax.lax.dot_general(
        w, x, (((1,), (1,)), ((), ())), preferred_element_type=jnp.float32
    )

    rows = jax.lax.broadcasted_iota(jnp.int32, logits_t.shape, 0)
    work = logits_t
    vals = []
    idxs = []
    for _ in range(_TOP_K):
        m = jnp.max(work, axis=0, keepdims=True)
        sel = jnp.min(
            jnp.where(work == m, rows, _NUM_EXPERTS), axis=0, keepdims=True
        )
        vals.append(m)
        idxs.append(sel)
        work = jnp.where(rows == sel, jnp.float32(-jnp.inf), work)

    top = jnp.concatenate(vals, axis=0)
    e = jnp.exp(top - top[0:1, :])
    probs_ref[...] = jnp.transpose(e / jnp.sum(e, axis=0, keepdims=True))
    idx_ref[...] = jnp.transpose(jnp.concatenate(idxs, axis=0))


@jax.jit
def kernel(input, W):
    n_tokens, d = input.shape
    n_exp = W.shape[0]
    grid = n_tokens // _TILE
    probs, indices = pl.pallas_call(
        _router_body,
        grid=(grid,),
        in_specs=[
            pl.BlockSpec((_TILE, d), lambda i: (i, 0)),
            pl.BlockSpec((n_exp, d), lambda i: (0, 0)),
        ],
        out_specs=[
            pl.BlockSpec((_TILE, _TOP_K), lambda i: (i, 0)),
            pl.BlockSpec((_TILE, _TOP_K), lambda i: (i, 0)),
        ],
        out_shape=[
            jax.ShapeDtypeStruct((n_tokens, _TOP_K), jnp.float32),
            jax.ShapeDtypeStruct((n_tokens, _TOP_K), jnp.int32),
        ],
    )(input, W)
    return probs, indices
